# baseline (speedup 1.0000x reference)
import functools

import jax
import jax.numpy as jnp
from jax import lax
from jax.experimental import pallas as pl
from jax.experimental.pallas import tpu as pltpu
from jax.experimental.pallas import tpu_sc as plsc

BATCH = 16384
HIST = 50
HPAD = 56
DIM = 64
DPAD = 128

NC = 2
NS = 16
NW = NC * NS

CHUNK = 128
TOTAL = BATCH * HIST
N_CHUNKS = TOTAL // CHUNK
CPW = N_CHUNKS // NW
NBUF = 4


def _cbow_body(idx_hbm, table_hbm, out_hbm, idx_v, rows_v, gsems, ssems):
    wid = lax.axis_index("s") * NC + lax.axis_index("c")
    chunk0 = wid * CPW

    pltpu.sync_copy(idx_hbm.at[pl.ds(chunk0, CPW)], idx_v)

    def fire(t, b):
        pltpu.async_copy(table_hbm.at[idx_v.at[t]], rows_v.at[b], gsems.at[b])

    def drain(b):
        pltpu.make_async_copy(table_hbm.at[pl.ds(0, CHUNK)], rows_v.at[b],
                              gsems.at[b]).wait()

    def store(t, b):
        c = chunk0 + t
        h = lax.div(c, CHUNK)
        bc = lax.rem(c, CHUNK)
        pltpu.async_copy(
            rows_v.at[b],
            out_hbm.at[pl.ds(bc * CHUNK, CHUNK), h, pl.ds(0, DIM)],
            ssems.at[b])

    def store_wait(b):
        pltpu.make_async_copy(rows_v.at[b],
                              out_hbm.at[pl.ds(0, CHUNK), 0, pl.ds(0, DIM)],
                              ssems.at[b]).wait()

    for b in range(NBUF):
        fire(b, b)

    def step(i, _):
        for b in range(NBUF):
            drain(b)
            store(i * NBUF + b, b)
        for b in range(NBUF):
            t = i * NBUF + b
            store_wait(b)

            @pl.when(t + NBUF < CPW)
            def _():
                fire(t + NBUF, b)
        return _

    lax.fori_loop(0, CPW // NBUF, step, None, unroll=False)


@functools.partial(jax.jit, static_argnames=())
def kernel(input_ids, table):
    idx = input_ids.astype(jnp.int32).T.reshape(N_CHUNKS, CHUNK)
    mesh = plsc.VectorSubcoreMesh(core_axis_name="c", subcore_axis_name="s",
                                  num_cores=NC, num_subcores=NS)
    outp = pl.kernel(
        _cbow_body,
        out_type=jax.ShapeDtypeStruct((BATCH, HPAD, DPAD), jnp.float32),
        mesh=mesh,
        scratch_types=[
            pltpu.VMEM((CPW, CHUNK), jnp.int32),
            pltpu.VMEM((NBUF, CHUNK, DIM), jnp.float32),
            pltpu.SemaphoreType.DMA((NBUF,)),
            pltpu.SemaphoreType.DMA((NBUF,)),
        ],
        compiler_params=pltpu.CompilerParams(use_tc_tiling_on_sc=False,
                                             needs_layout_passes=False),
    )(idx, table)
    return outp[:, :HIST, :DIM]
